# baseline (speedup 1.0000x reference)
import jax
import jax.numpy as jnp
from jax.experimental import pallas as pl
from jax.experimental.pallas import tpu as pltpu

B = 4
N = 10000
E = 320000
D_V = 128
D_E = 16
D_C = 128
D_OUT = 128

G = 10
V_C = N // G
E_C = (E * D_E // 128) // G


def _body(ctx_ref, v_ref, e_ref, w_ref, b_ref, out_ref, acc_v, acc_e):
    i = pl.program_id(0)

    @pl.when(i == 0)
    def _init():
        acc_v[...] = jnp.zeros_like(acc_v)
        acc_e[...] = jnp.zeros_like(acc_e)

    acc_v[...] += jnp.sum(v_ref[...], axis=1)
    acc_e[...] += jnp.sum(e_ref[...], axis=1)

    @pl.when(i == pl.num_programs(0) - 1)
    def _final():
        v_agg = acc_v[...] * (1.0 / N)
        e_sum = acc_e[...]
        e_agg = jnp.zeros((B, 16), jnp.float32)
        for k in range(8):
            e_agg = e_agg + e_sum[:, k * 16:(k + 1) * 16]
        e_agg = e_agg * (1.0 / E)
        out = (
            jnp.dot(ctx_ref[...], w_ref[0:D_C], preferred_element_type=jnp.float32)
            + jnp.dot(v_agg, w_ref[D_C:D_C + D_V], preferred_element_type=jnp.float32)
            + jnp.dot(e_agg, w_ref[D_C + D_V:D_C + D_V + D_E],
                      preferred_element_type=jnp.float32)
            + b_ref[...]
        )
        out_ref[...] = out


def kernel(context, vertex, edge, W, b):
    ctx = context.reshape(B, D_C)
    edge_r = edge.reshape(B, E * D_E // 128, 128)
    b_r = b.reshape(1, D_OUT)

    out = pl.pallas_call(
        _body,
        grid=(G,),
        in_specs=[
            pl.BlockSpec((B, D_C), lambda i: (0, 0)),
            pl.BlockSpec((B, V_C, D_V), lambda i: (0, i, 0)),
            pl.BlockSpec((B, E_C, 128), lambda i: (0, i, 0)),
            pl.BlockSpec((D_C + D_V + D_E, D_OUT), lambda i: (0, 0)),
            pl.BlockSpec((1, D_OUT), lambda i: (0, 0)),
        ],
        out_specs=pl.BlockSpec((B, D_OUT), lambda i: (0, 0)),
        out_shape=jax.ShapeDtypeStruct((B, D_OUT), jnp.float32),
        scratch_shapes=[
            pltpu.VMEM((B, 128), jnp.float32),
            pltpu.VMEM((B, 128), jnp.float32),
        ],
    )(ctx, vertex, edge_r, W, b_r)
    return out.reshape(B, 1, D_OUT)


# speedup vs baseline: 1.0023x; 1.0023x over previous
import jax
import jax.numpy as jnp
from jax.experimental import pallas as pl
from jax.experimental.pallas import tpu as pltpu

B = 4
N = 10000
E = 320000
D_V = 128
D_E = 16
D_C = 128
D_OUT = 128

G = 10
V_C = N // G
E_C = (E * D_E // 128) // G


def _body(ctx_ref, v_ref, e_ref, w_ref, b_ref, out_ref, acc_v, acc_e):
    i = pl.program_id(0)

    @pl.when(i == 0)
    def _init():
        acc_v[...] = jnp.zeros_like(acc_v)
        acc_e[...] = jnp.zeros_like(acc_e)

    acc_v[...] += jnp.sum(v_ref[...].reshape(B, V_C // 8, 8, D_V), axis=1)
    acc_e[...] += jnp.sum(e_ref[...].reshape(B, E_C // 8, 8, 128), axis=1)

    @pl.when(i == pl.num_programs(0) - 1)
    def _final():
        v_agg = jnp.sum(acc_v[...], axis=1) * (1.0 / N)
        e_sum = jnp.sum(acc_e[...], axis=1)
        e_agg = jnp.zeros((B, 16), jnp.float32)
        for k in range(8):
            e_agg = e_agg + e_sum[:, k * 16:(k + 1) * 16]
        e_agg = e_agg * (1.0 / E)
        out = (
            jnp.dot(ctx_ref[...], w_ref[0:D_C], preferred_element_type=jnp.float32)
            + jnp.dot(v_agg, w_ref[D_C:D_C + D_V], preferred_element_type=jnp.float32)
            + jnp.dot(e_agg, w_ref[D_C + D_V:D_C + D_V + D_E],
                      preferred_element_type=jnp.float32)
            + b_ref[...]
        )
        out_ref[...] = out


def kernel(context, vertex, edge, W, b):
    ctx = context.reshape(B, D_C)
    edge_r = edge.reshape(B, E * D_E // 128, 128)
    b_r = b.reshape(1, D_OUT)

    out = pl.pallas_call(
        _body,
        grid=(G,),
        in_specs=[
            pl.BlockSpec((B, D_C), lambda i: (0, 0)),
            pl.BlockSpec((B, V_C, D_V), lambda i: (0, i, 0)),
            pl.BlockSpec((B, E_C, 128), lambda i: (0, i, 0)),
            pl.BlockSpec((D_C + D_V + D_E, D_OUT), lambda i: (0, 0)),
            pl.BlockSpec((1, D_OUT), lambda i: (0, 0)),
        ],
        out_specs=pl.BlockSpec((B, D_OUT), lambda i: (0, 0)),
        out_shape=jax.ShapeDtypeStruct((B, D_OUT), jnp.float32),
        scratch_shapes=[
            pltpu.VMEM((B, 8, 128), jnp.float32),
            pltpu.VMEM((B, 8, 128), jnp.float32),
        ],
    )(ctx, vertex, edge_r, W, b_r)
    return out.reshape(B, 1, D_OUT)
